# baseline (speedup 1.0000x reference)
import functools

import jax
import jax.numpy as jnp
from jax import lax
from jax.experimental import pallas as pl
from jax.experimental.pallas import tpu as pltpu
from jax.experimental.pallas import tpu_sc as plsc

_LANES = 16


def _score_body(w_ref, g_ref, o_ref, *, n, block, off_blocks):
    i = pl.program_id(0)
    w = w_ref[...]
    g = g_ref[...]
    dots = jnp.dot(w, g, preferred_element_type=jnp.float32)
    sq = jnp.dot(w, g * g, preferred_element_type=jnp.float32)
    dot = dots[0:1, :]
    gn2 = sq[1:2, :]
    tn2 = w[2, 0]
    denom = jnp.maximum(gn2 * tn2, jnp.float32(1e-16))
    s = (dot * jnp.abs(dot) / denom).reshape(1, 1, block)
    col = lax.broadcasted_iota(jnp.int32, (1, 1, block), 2)
    valid = ((i + off_blocks) * block + col) < n
    o_ref[...] = jnp.where(valid, s, jnp.float32(-jnp.inf))


def _score_weights(text_embedding, d):
    tn2 = jnp.sum(text_embedding * text_embedding, axis=-1, keepdims=True)
    return jnp.concatenate(
        [
            text_embedding,
            jnp.ones((1, d), jnp.float32),
            jnp.pad(tn2, ((0, 0), (0, d - 1))),
            jnp.zeros((5, d), jnp.float32),
        ],
        axis=0,
    )


def _scores(w, gt, block, nb, off_blocks, n):
    d = gt.shape[0]
    return pl.pallas_call(
        functools.partial(_score_body, n=n, block=block,
                          off_blocks=off_blocks),
        grid=(nb,),
        in_specs=[
            pl.BlockSpec((8, d), lambda i: (0, 0)),
            pl.BlockSpec((d, block), lambda i: (0, i + off_blocks)),
        ],
        out_specs=pl.BlockSpec((1, 1, block), lambda i: (i, 0, 0)),
        out_shape=jax.ShapeDtypeStruct((nb, 1, block), jnp.float32),
        compiler_params=pltpu.CompilerParams(
            dimension_semantics=("arbitrary",)),
    )(w, gt)


def _merge_sorted(top_v, top_i, sv, si):
    rv = lax.rev(sv, (0,))
    ri = lax.rev(si, (0,))
    m = top_v >= rv
    mv = jnp.where(m, top_v, rv)
    mi = jnp.where(m, top_i, ri)
    nv, ni = plsc.sort_key_val(mv, mi)
    return nv, ni


def _stage1_body(scores_hbm, vals_out, idxs_out, buf, candv, candi,
                 stage_v, stage_i, *, chunk, num_cores, glob_off):
    c = lax.axis_index("c")
    s = lax.axis_index("s")
    wid = s * num_cores + c
    base = wid * chunk
    pltpu.sync_copy(scores_hbm.at[pl.ds(base, chunk)], buf)
    lane = lax.iota(jnp.int32, _LANES)
    nvreg = chunk // _LANES
    ninf = jnp.full((_LANES,), -jnp.inf, jnp.float32)

    ua = 8
    def amax(i, m):
        for k in range(ua):
            m = jnp.maximum(m, buf[pl.ds((i * ua + k) * _LANES, _LANES)])
        return m

    lane_max = lax.fori_loop(0, nvreg // ua, amax, ninf)
    tau = jnp.full((_LANES,), 1.0, jnp.float32) * jnp.min(lane_max)

    ub = 1
    def bbody(i, ptr):
        for k in range(ub):
            v = buf[pl.ds((i * ub + k) * _LANES, _LANES)]
            iv = glob_off + base + (i * ub + k) * _LANES + lane
            mask = v >= tau
            plsc.store_compressed(candv.at[pl.ds(ptr, _LANES)], v, mask=mask)
            plsc.store_compressed(candi.at[pl.ds(ptr, _LANES)], iv, mask=mask)
            cnt = plsc.all_reduce_population_count(mask)
            ptr = ptr + (cnt if cnt.ndim == 0 else cnt[0])
        return ptr

    nc = lax.fori_loop(0, nvreg // ub, bbody, jnp.int32(0))
    candv[pl.ds(nc, _LANES)] = ninf

    def mbody(j, carry):
        tv, ti = carry
        sv, si = plsc.sort_key_val(candv[pl.ds(j * _LANES, _LANES)],
                                   candi[pl.ds(j * _LANES, _LANES)])
        return _merge_sorted(tv, ti, sv, si)

    init = (ninf, jnp.zeros((_LANES,), jnp.int32))
    top_v, top_i = lax.fori_loop(0, (nc + _LANES - 1) // _LANES, mbody, init)
    stage_v[...] = top_v
    stage_i[...] = top_i
    pltpu.sync_copy(stage_v, vals_out.at[wid])
    pltpu.sync_copy(stage_i, idxs_out.at[wid])


def _stage2_body(vals_hbm, idxs_hbm, out_hbm, vbuf, ibuf, obuf, *,
                 workers, num_cores):
    c = lax.axis_index("c")
    s = lax.axis_index("s")
    wid = s * num_cores + c

    @pl.when(wid == 0)
    def _():
        pltpu.sync_copy(vals_hbm, vbuf)
        pltpu.sync_copy(idxs_hbm, ibuf)

        def body(j, carry):
            top_v, top_i = carry
            sv = vbuf[pl.ds(j * _LANES, _LANES)]
            si = ibuf[pl.ds(j * _LANES, _LANES)]
            return _merge_sorted(top_v, top_i, sv, si)

        init = (jnp.full((_LANES,), -jnp.inf, jnp.float32),
                jnp.zeros((_LANES,), jnp.int32))
        top_v, top_i = lax.fori_loop(0, workers, body, init)
        obuf[...] = lax.rev(top_i, (0,))
        pltpu.sync_copy(obuf, out_hbm)


def _stage1_call(scores_flat, glob_off, num_cores, workers, mesh):
    total = scores_flat.shape[0]
    chunk = total // workers
    return pl.kernel(
        functools.partial(_stage1_body, chunk=chunk, num_cores=num_cores,
                          glob_off=glob_off),
        mesh=mesh,
        out_type=[
            jax.ShapeDtypeStruct((workers, _LANES), jnp.float32),
            jax.ShapeDtypeStruct((workers, _LANES), jnp.int32),
        ],
        scratch_types=[
            pltpu.VMEM((chunk,), jnp.float32),
            pltpu.VMEM((chunk + _LANES,), jnp.float32),
            pltpu.VMEM((chunk + _LANES,), jnp.int32),
            pltpu.VMEM((_LANES,), jnp.float32),
            pltpu.VMEM((_LANES,), jnp.int32),
        ],
        compiler_params=pltpu.CompilerParams(needs_layout_passes=False),
    )(scores_flat)


def kernel(text_embedding, graph_embedding, top_k):
    del top_k
    n, d = graph_embedding.shape
    block = 65536
    nb = pl.cdiv(n, block)
    nb_a = (3 * nb) // 4
    nb_b = nb - nb_a

    info = plsc.get_sparse_core_info()
    num_cores, num_subcores = info.num_cores, info.num_subcores
    workers = num_cores * num_subcores
    mesh = plsc.VectorSubcoreMesh(core_axis_name="c", subcore_axis_name="s")

    gt = graph_embedding.T
    w = _score_weights(text_embedding, d)

    scores_a = _scores(w, gt, block, nb_a, 0, n)
    va, ia = _stage1_call(scores_a.reshape(-1), 0, num_cores, workers, mesh)
    scores_b = _scores(w, gt, block, nb_b, nb_a, n)
    vb, ib = _stage1_call(scores_b.reshape(-1), nb_a * block,
                          num_cores, workers, mesh)

    vals = jnp.concatenate([va, vb], axis=0)
    idxs = jnp.concatenate([ia, ib], axis=0)
    rows = 2 * workers
    out16 = pl.kernel(
        functools.partial(_stage2_body, workers=rows, num_cores=num_cores),
        mesh=mesh,
        out_type=jax.ShapeDtypeStruct((_LANES,), jnp.int32),
        scratch_types=[
            pltpu.VMEM((rows * _LANES,), jnp.float32),
            pltpu.VMEM((rows * _LANES,), jnp.int32),
            pltpu.VMEM((_LANES,), jnp.int32),
        ],
        compiler_params=pltpu.CompilerParams(needs_layout_passes=False),
    )(vals.reshape(-1), idxs.reshape(-1))
    return out16[:10]


# speedup vs baseline: 1.0711x; 1.0711x over previous
import functools

import jax
import jax.numpy as jnp
from jax import lax
from jax.experimental import pallas as pl
from jax.experimental.pallas import tpu as pltpu
from jax.experimental.pallas import tpu_sc as plsc

_LANES = 16


def _score_body(w_ref, g_ref, o_ref, *, n, block, off_blocks):
    i = pl.program_id(0)
    w = w_ref[...]
    g = g_ref[...]
    dots = jnp.dot(w, g, preferred_element_type=jnp.float32)
    sq = jnp.dot(w, g * g, preferred_element_type=jnp.float32)
    dot = dots[0:1, :]
    gn2 = sq[1:2, :]
    tn2 = w[2, 0]
    denom = jnp.maximum(gn2 * tn2, jnp.float32(1e-16))
    s = (dot * jnp.abs(dot) / denom).reshape(1, 1, block)
    col = lax.broadcasted_iota(jnp.int32, (1, 1, block), 2)
    valid = ((i + off_blocks) * block + col) < n
    o_ref[...] = jnp.where(valid, s, jnp.float32(-jnp.inf))


def _score_weights(text_embedding, d):
    tn2 = jnp.sum(text_embedding * text_embedding, axis=-1, keepdims=True)
    return jnp.concatenate(
        [
            text_embedding,
            jnp.ones((1, d), jnp.float32),
            jnp.pad(tn2, ((0, 0), (0, d - 1))),
            jnp.zeros((5, d), jnp.float32),
        ],
        axis=0,
    )


def _scores(w, gt, block, nb, off_blocks, n):
    d = gt.shape[0]
    return pl.pallas_call(
        functools.partial(_score_body, n=n, block=block,
                          off_blocks=off_blocks),
        grid=(nb,),
        in_specs=[
            pl.BlockSpec((8, d), lambda i: (0, 0)),
            pl.BlockSpec((d, block), lambda i: (0, i + off_blocks)),
        ],
        out_specs=pl.BlockSpec((1, 1, block), lambda i: (i, 0, 0)),
        out_shape=jax.ShapeDtypeStruct((nb, 1, block), jnp.float32),
        compiler_params=pltpu.CompilerParams(
            dimension_semantics=("arbitrary",)),
    )(w, gt)


def _merge_sorted(top_v, top_i, sv, si):
    rv = lax.rev(sv, (0,))
    ri = lax.rev(si, (0,))
    m = top_v >= rv
    mv = jnp.where(m, top_v, rv)
    mi = jnp.where(m, top_i, ri)
    nv, ni = plsc.sort_key_val(mv, mi)
    return nv, ni


def _stage1_body(scores_hbm, vals_out, idxs_out, buf,
                 stage_v, stage_i, *, chunk, num_cores, glob_off):
    c = lax.axis_index("c")
    s = lax.axis_index("s")
    wid = s * num_cores + c
    base = wid * chunk
    pltpu.sync_copy(scores_hbm.at[pl.ds(base, chunk)], buf)
    lane = lax.iota(jnp.int32, _LANES)
    ninf = jnp.full((_LANES,), -jnp.inf, jnp.float32)

    def body(i, carry):
        top_v, top_i = carry
        v = buf[pl.ds(i * _LANES, _LANES)]
        iv = glob_off + base + i * _LANES + lane
        sv, si = plsc.sort_key_val(v, iv)
        return _merge_sorted(top_v, top_i, sv, si)

    init = (ninf, jnp.zeros((_LANES,), jnp.int32))
    top_v, top_i = lax.fori_loop(0, chunk // _LANES, body, init)
    stage_v[...] = top_v
    stage_i[...] = top_i
    pltpu.sync_copy(stage_v, vals_out.at[wid])
    pltpu.sync_copy(stage_i, idxs_out.at[wid])


def _stage2_body(vals_hbm, idxs_hbm, out_hbm, vbuf, ibuf, obuf, *,
                 workers, num_cores):
    c = lax.axis_index("c")
    s = lax.axis_index("s")
    wid = s * num_cores + c

    @pl.when(wid == 0)
    def _():
        pltpu.sync_copy(vals_hbm, vbuf)
        pltpu.sync_copy(idxs_hbm, ibuf)

        def body(j, carry):
            top_v, top_i = carry
            sv = vbuf[pl.ds(j * _LANES, _LANES)]
            si = ibuf[pl.ds(j * _LANES, _LANES)]
            return _merge_sorted(top_v, top_i, sv, si)

        init = (jnp.full((_LANES,), -jnp.inf, jnp.float32),
                jnp.zeros((_LANES,), jnp.int32))
        top_v, top_i = lax.fori_loop(0, workers, body, init)
        obuf[...] = lax.rev(top_i, (0,))
        pltpu.sync_copy(obuf, out_hbm)


def _stage1_call(scores_flat, glob_off, num_cores, workers, mesh):
    total = scores_flat.shape[0]
    chunk = total // workers
    return pl.kernel(
        functools.partial(_stage1_body, chunk=chunk, num_cores=num_cores,
                          glob_off=glob_off),
        mesh=mesh,
        out_type=[
            jax.ShapeDtypeStruct((workers, _LANES), jnp.float32),
            jax.ShapeDtypeStruct((workers, _LANES), jnp.int32),
        ],
        scratch_types=[
            pltpu.VMEM((chunk,), jnp.float32),
            pltpu.VMEM((_LANES,), jnp.float32),
            pltpu.VMEM((_LANES,), jnp.int32),
        ],
        compiler_params=pltpu.CompilerParams(needs_layout_passes=False),
    )(scores_flat)


def kernel(text_embedding, graph_embedding, top_k):
    del top_k
    n, d = graph_embedding.shape
    block = 32768
    nb = pl.cdiv(n, block)
    nb_a = (3 * nb) // 4
    nb_b = nb - nb_a

    info = plsc.get_sparse_core_info()
    num_cores, num_subcores = info.num_cores, info.num_subcores
    workers = num_cores * num_subcores
    mesh = plsc.VectorSubcoreMesh(core_axis_name="c", subcore_axis_name="s")

    gt = graph_embedding.T
    w = _score_weights(text_embedding, d)

    scores_a = _scores(w, gt, block, nb_a, 0, n)
    va, ia = _stage1_call(scores_a.reshape(-1), 0, num_cores, workers, mesh)
    scores_b = _scores(w, gt, block, nb_b, nb_a, n)
    vb, ib = _stage1_call(scores_b.reshape(-1), nb_a * block,
                          num_cores, workers, mesh)

    vals = jnp.concatenate([va, vb], axis=0)
    idxs = jnp.concatenate([ia, ib], axis=0)
    rows = 2 * workers
    out16 = pl.kernel(
        functools.partial(_stage2_body, workers=rows, num_cores=num_cores),
        mesh=mesh,
        out_type=jax.ShapeDtypeStruct((_LANES,), jnp.int32),
        scratch_types=[
            pltpu.VMEM((rows * _LANES,), jnp.float32),
            pltpu.VMEM((rows * _LANES,), jnp.int32),
            pltpu.VMEM((_LANES,), jnp.int32),
        ],
        compiler_params=pltpu.CompilerParams(needs_layout_passes=False),
    )(vals.reshape(-1), idxs.reshape(-1))
    return out16[:10]
